# initial kernel scaffold (unmeasured)
import jax
import jax.numpy as jnp
from jax import lax
from jax.experimental import pallas as pl
from jax.experimental.pallas import tpu as pltpu

N_DEV = 32
M = 4096
N = 2048
CH = M // N_DEV

_GELU_C = 0.7978845608028654


def _gelu(y):
    return 0.5 * y * (1.0 + jnp.tanh(_GELU_C * (y + 0.044715 * y * y * y)))


def kernel(x, w_mat):
    x = x.astype(jnp.bfloat16)
    w_mat = w_mat.astype(jnp.bfloat16)

    def body(x_ref, w_ref, out_ref, comm_ref, send_sems, recv_sems):
        my = lax.axis_index("i")
        left = lax.rem(my + N_DEV - 1, N_DEV)
        right = lax.rem(my + 1, N_DEV)

        barrier_sem = pltpu.get_barrier_semaphore()
        pl.semaphore_signal(barrier_sem, inc=1, device_id=(left,),
                            device_id_type=pl.DeviceIdType.MESH)
        pl.semaphore_signal(barrier_sem, inc=1, device_id=(right,),
                            device_id_type=pl.DeviceIdType.MESH)
        pl.semaphore_wait(barrier_sem, 2)

        def partial_chunk(c):
            xa = x_ref[pl.ds(c * CH, CH), :]
            return jnp.dot(xa, w_ref[:, :], preferred_element_type=jnp.float32)

        def hop(h, target_slot_value=None):
            send_slot = h % 2
            recv_slot = (h + 1) % 2
            if target_slot_value is not None:
                comm_ref[send_slot] = target_slot_value
            rdma = pltpu.make_async_remote_copy(
                src_ref=comm_ref.at[send_slot],
                dst_ref=comm_ref.at[recv_slot],
                send_sem=send_sems.at[send_slot],
                recv_sem=recv_sems.at[recv_slot],
                device_id=(right,),
                device_id_type=pl.DeviceIdType.MESH,
            )
            rdma.start()
            rdma.wait()
            return recv_slot

        comm_ref[0] = partial_chunk(my).astype(jnp.bfloat16)
        acc = None
        for s in range(N_DEV - 1):
            recv_slot = hop(s)
            c = lax.rem(my + N_DEV - s - 1, N_DEV)
            summed = comm_ref[recv_slot].astype(jnp.float32) + partial_chunk(c)
            if s < N_DEV - 2:
                comm_ref[recv_slot] = summed.astype(jnp.bfloat16)
            else:
                acc = summed

        own = lax.rem(my + 1, N_DEV)
        g = _gelu(acc)
        out_ref[pl.ds(own * CH, CH), :] = g

        for t in range(N_DEV - 1):
            h = (N_DEV - 1) + t
            recv_slot = hop(h, target_slot_value=g.astype(jnp.bfloat16) if t == 0 else None)
            c = lax.rem(my + N_DEV - t, N_DEV)
            out_ref[pl.ds(c * CH, CH), :] = comm_ref[recv_slot].astype(jnp.float32)

    return pl.pallas_call(
        body,
        out_shape=jax.ShapeDtypeStruct((M, N), jnp.float32),
        in_specs=[
            pl.BlockSpec(memory_space=pltpu.VMEM),
            pl.BlockSpec(memory_space=pltpu.VMEM),
        ],
        out_specs=pl.BlockSpec(memory_space=pltpu.VMEM),
        scratch_shapes=[
            pltpu.VMEM((2, CH, N), jnp.bfloat16),
            pltpu.SemaphoreType.DMA((2,)),
            pltpu.SemaphoreType.DMA((2,)),
        ],
        compiler_params=pltpu.CompilerParams(collective_id=0),
    )(x, w_mat)


# baseline (device time: 486460 ns/iter reference)
import jax
import jax.numpy as jnp
from jax import lax
from jax.experimental import pallas as pl
from jax.experimental.pallas import tpu as pltpu

N_DEV = 32
M = 4096
N = 2048
CH = M // N_DEV

_GELU_C = 0.7978845608028654


def _gelu(y):
    return 0.5 * y * (1.0 + jnp.tanh(_GELU_C * (y + 0.044715 * y * y * y)))


def kernel(x, w_mat):
    x = x.astype(jnp.bfloat16)
    w_mat = w_mat.astype(jnp.bfloat16)

    def body(x_ref, w_ref, out_ref, comm_ref, send_sems, recv_sems):
        my = lax.axis_index("i")
        left = lax.rem(my + N_DEV - 1, N_DEV)
        right = lax.rem(my + 1, N_DEV)

        barrier_sem = pltpu.get_barrier_semaphore()
        pl.semaphore_signal(barrier_sem, inc=1, device_id=(left,),
                            device_id_type=pl.DeviceIdType.MESH)
        pl.semaphore_signal(barrier_sem, inc=1, device_id=(right,),
                            device_id_type=pl.DeviceIdType.MESH)
        pl.semaphore_wait(barrier_sem, 2)

        def partial_chunk(c):
            xa = x_ref[pl.ds(c * CH, CH), :]
            return jnp.dot(xa, w_ref[:, :], preferred_element_type=jnp.float32)

        def hop(h, target_slot_value=None):
            send_slot = h % 2
            recv_slot = (h + 1) % 2
            if target_slot_value is not None:
                comm_ref[send_slot] = target_slot_value
            rdma = pltpu.make_async_remote_copy(
                src_ref=comm_ref.at[send_slot],
                dst_ref=comm_ref.at[recv_slot],
                send_sem=send_sems.at[send_slot],
                recv_sem=recv_sems.at[recv_slot],
                device_id=(right,),
                device_id_type=pl.DeviceIdType.MESH,
            )
            rdma.start()
            rdma.wait()
            return recv_slot

        comm_ref[0] = partial_chunk(my).astype(jnp.bfloat16)
        acc = None
        for s in range(N_DEV - 1):
            recv_slot = hop(s)
            c = lax.rem(my + N_DEV - s - 1, N_DEV)
            summed = comm_ref[recv_slot].astype(jnp.float32) + partial_chunk(c)
            if s < N_DEV - 2:
                comm_ref[recv_slot] = summed.astype(jnp.bfloat16)
            else:
                acc = summed

        own = lax.rem(my + 1, N_DEV)
        g = _gelu(acc).astype(jnp.bfloat16)
        out_ref[pl.ds(own * CH, CH), :] = g

        for t in range(N_DEV - 1):
            h = (N_DEV - 1) + t
            recv_slot = hop(h, target_slot_value=g if t == 0 else None)
            c = lax.rem(my + N_DEV - t, N_DEV)
            out_ref[pl.ds(c * CH, CH), :] = comm_ref[recv_slot]

    return pl.pallas_call(
        body,
        out_shape=jax.ShapeDtypeStruct((M, N), jnp.bfloat16),
        in_specs=[
            pl.BlockSpec(memory_space=pltpu.VMEM),
            pl.BlockSpec(memory_space=pltpu.VMEM),
        ],
        out_specs=pl.BlockSpec(memory_space=pltpu.VMEM),
        scratch_shapes=[
            pltpu.VMEM((2, CH, N), jnp.bfloat16),
            pltpu.SemaphoreType.DMA((2,)),
            pltpu.SemaphoreType.DMA((2,)),
        ],
        compiler_params=pltpu.CompilerParams(collective_id=0),
    )(x, w_mat)


# device time: 455867 ns/iter; 1.0671x vs baseline; 1.0671x over previous
import jax
import jax.numpy as jnp
from jax import lax
from jax.experimental import pallas as pl
from jax.experimental.pallas import tpu as pltpu

N_DEV = 32
M = 4096
N = 2048
CH = M // N_DEV
NH = N // 2

_GELU_C = 0.7978845608028654


def _gelu(y):
    return 0.5 * y * (1.0 + jnp.tanh(_GELU_C * (y + 0.044715 * y * y * y)))


def kernel(x, w_mat):
    x = x.astype(jnp.bfloat16)
    w_mat = w_mat.astype(jnp.bfloat16)

    def body(x_ref, w_ref, out_ref,
             comm_p, comm_m, ssem_p, rsem_p, ssem_m, rsem_m):
        my = lax.axis_index("i")
        left = lax.rem(my + N_DEV - 1, N_DEV)
        right = lax.rem(my + 1, N_DEV)

        barrier_sem = pltpu.get_barrier_semaphore()
        pl.semaphore_signal(barrier_sem, inc=1, device_id=(left,),
                            device_id_type=pl.DeviceIdType.MESH)
        pl.semaphore_signal(barrier_sem, inc=1, device_id=(right,),
                            device_id_type=pl.DeviceIdType.MESH)
        pl.semaphore_wait(barrier_sem, 2)

        def partial_chunk(c, col0):
            xa = x_ref[pl.ds(c * CH, CH), :]
            wa = w_ref[:, pl.ds(col0, NH)]
            return jnp.dot(xa, wa, preferred_element_type=jnp.float32)

        def make_hop(comm, ssem, rsem, target):
            def hop(h):
                send_slot = h % 2
                recv_slot = (h + 1) % 2
                rdma = pltpu.make_async_remote_copy(
                    src_ref=comm.at[send_slot],
                    dst_ref=comm.at[recv_slot],
                    send_sem=ssem.at[send_slot],
                    recv_sem=rsem.at[recv_slot],
                    device_id=(target,),
                    device_id_type=pl.DeviceIdType.MESH,
                )
                rdma.start()
                return rdma, recv_slot
            return hop

        hop_p = make_hop(comm_p, ssem_p, rsem_p, right)
        hop_m = make_hop(comm_m, ssem_m, rsem_m, left)

        comm_p[0] = partial_chunk(my, 0).astype(jnp.bfloat16)
        comm_m[0] = partial_chunk(my, NH).astype(jnp.bfloat16)
        acc_p = acc_m = None
        for s in range(N_DEV - 1):
            rdma_p, slot_p = hop_p(s)
            rdma_m, slot_m = hop_m(s)
            c_p = lax.rem(my + N_DEV - s - 1, N_DEV)
            c_m = lax.rem(my + s + 1, N_DEV)
            rdma_p.wait()
            sum_p = comm_p[slot_p].astype(jnp.float32) + partial_chunk(c_p, 0)
            if s < N_DEV - 2:
                comm_p[slot_p] = sum_p.astype(jnp.bfloat16)
            else:
                acc_p = sum_p
            rdma_m.wait()
            sum_m = comm_m[slot_m].astype(jnp.float32) + partial_chunk(c_m, NH)
            if s < N_DEV - 2:
                comm_m[slot_m] = sum_m.astype(jnp.bfloat16)
            else:
                acc_m = sum_m

        own_p = lax.rem(my + 1, N_DEV)
        own_m = lax.rem(my + N_DEV - 1, N_DEV)
        g_p = _gelu(acc_p).astype(jnp.bfloat16)
        g_m = _gelu(acc_m).astype(jnp.bfloat16)
        out_ref[pl.ds(own_p * CH, CH), pl.ds(0, NH)] = g_p
        out_ref[pl.ds(own_m * CH, CH), pl.ds(NH, NH)] = g_m

        comm_p[1] = g_p
        comm_m[1] = g_m
        for t in range(N_DEV - 1):
            h = (N_DEV - 1) + t
            rdma_p, slot_p = hop_p(h)
            rdma_m, slot_m = hop_m(h)
            c_p = lax.rem(my + N_DEV - t, N_DEV)
            c_m = lax.rem(my + t, N_DEV)
            rdma_p.wait()
            out_ref[pl.ds(c_p * CH, CH), pl.ds(0, NH)] = comm_p[slot_p]
            rdma_m.wait()
            out_ref[pl.ds(c_m * CH, CH), pl.ds(NH, NH)] = comm_m[slot_m]

    return pl.pallas_call(
        body,
        out_shape=jax.ShapeDtypeStruct((M, N), jnp.bfloat16),
        in_specs=[
            pl.BlockSpec(memory_space=pltpu.VMEM),
            pl.BlockSpec(memory_space=pltpu.VMEM),
        ],
        out_specs=pl.BlockSpec(memory_space=pltpu.VMEM),
        scratch_shapes=[
            pltpu.VMEM((2, CH, NH), jnp.bfloat16),
            pltpu.VMEM((2, CH, NH), jnp.bfloat16),
            pltpu.SemaphoreType.DMA((2,)),
            pltpu.SemaphoreType.DMA((2,)),
            pltpu.SemaphoreType.DMA((2,)),
            pltpu.SemaphoreType.DMA((2,)),
        ],
        compiler_params=pltpu.CompilerParams(collective_id=0),
    )(x, w_mat)


# device time: 369065 ns/iter; 1.3181x vs baseline; 1.2352x over previous
import jax
import jax.numpy as jnp
from jax import lax
from jax.experimental import pallas as pl
from jax.experimental.pallas import tpu as pltpu

N_DEV = 32
M = 4096
N = 2048
CH = M // N_DEV
NH = N // 2
R = 4
S = 4
SUBW = NH // R
NRINGS = 2 * R
NSTEPS = 2 * (N_DEV - 1)

_GELU_C = 0.7978845608028654


def _gelu(y):
    return 0.5 * y * (1.0 + jnp.tanh(_GELU_C * (y + 0.044715 * y * y * y)))


def kernel(x, w_mat):
    x = x.astype(jnp.bfloat16)
    w_mat = w_mat.astype(jnp.bfloat16)

    def body(x_ref, w_ref, out_ref, comm, ssem, rsem, credit):
        my = lax.axis_index("i")
        left = lax.rem(my + N_DEV - 1, N_DEV)
        right = lax.rem(my + 1, N_DEV)

        rings = []
        for r in range(R):
            rings.append((2 * r, True, r * SUBW))
            rings.append((2 * r + 1, False, NH + r * SUBW))

        barrier_sem = pltpu.get_barrier_semaphore()
        pl.semaphore_signal(barrier_sem, inc=1, device_id=(left,),
                            device_id_type=pl.DeviceIdType.MESH)
        pl.semaphore_signal(barrier_sem, inc=1, device_id=(right,),
                            device_id_type=pl.DeviceIdType.MESH)
        pl.semaphore_wait(barrier_sem, 2)

        def partial_dir(c, col0):
            xa = x_ref[pl.ds(c * CH, CH), :]
            wa = w_ref[:, pl.ds(col0, NH)]
            return jnp.dot(xa, wa, preferred_element_type=jnp.float32)

        def desc(i, plus, h):
            return pltpu.make_async_remote_copy(
                src_ref=comm.at[i, h % S],
                dst_ref=comm.at[i, (h + 1) % S],
                send_sem=ssem.at[i, h % S],
                recv_sem=rsem.at[i, (h + 1) % S],
                device_id=(right if plus else left,),
                device_id_type=pl.DeviceIdType.MESH,
            )

        p0 = partial_dir(my, 0).astype(jnp.bfloat16)
        m0 = partial_dir(my, NH).astype(jnp.bfloat16)
        for i, plus, colbase in rings:
            src = p0 if plus else m0
            cb = colbase if plus else colbase - NH
            comm[i, 0] = src[:, cb:cb + SUBW]
            desc(i, plus, 0).start()

        for h in range(NSTEPS):
            if h <= N_DEV - 2:
                c_p = lax.rem(my + N_DEV - h - 1, N_DEV)
                c_m = lax.rem(my + h + 1, N_DEV)
                P_p = partial_dir(c_p, 0)
                P_m = partial_dir(c_m, NH)
            for i, plus, colbase in rings:
                d = desc(i, plus, h)
                slot = (h + 1) % S
                d.wait_recv()
                if h < N_DEV - 2:
                    P = P_p if plus else P_m
                    cb = colbase if plus else colbase - NH
                    summed = (comm[i, slot].astype(jnp.float32)
                              + P[:, cb:cb + SUBW])
                    comm[i, slot] = summed.astype(jnp.bfloat16)
                elif h == N_DEV - 2:
                    P = P_p if plus else P_m
                    cb = colbase if plus else colbase - NH
                    summed = (comm[i, slot].astype(jnp.float32)
                              + P[:, cb:cb + SUBW])
                    g = _gelu(summed).astype(jnp.bfloat16)
                    comm[i, slot] = g
                    own = lax.rem(my + 1, N_DEV) if plus else lax.rem(my + N_DEV - 1, N_DEV)
                    out_ref[pl.ds(own * CH, CH), pl.ds(colbase, SUBW)] = g
                if h < NSTEPS - 1:
                    k = h + 1
                    if k >= S - 1:
                        pl.semaphore_wait(credit.at[i], 1)
                    desc(i, plus, k).start()
                if h > N_DEV - 2:
                    t = h - (N_DEV - 1)
                    c = lax.rem(my + N_DEV - t, N_DEV) if plus else lax.rem(my + t, N_DEV)
                    out_ref[pl.ds(c * CH, CH), pl.ds(colbase, SUBW)] = comm[i, slot]
                d.wait_send()
                if h <= NSTEPS - 1 - (S - 1):
                    pl.semaphore_signal(
                        credit.at[i], inc=1,
                        device_id=(left if plus else right,),
                        device_id_type=pl.DeviceIdType.MESH,
                    )

    return pl.pallas_call(
        body,
        out_shape=jax.ShapeDtypeStruct((M, N), jnp.bfloat16),
        in_specs=[
            pl.BlockSpec(memory_space=pltpu.VMEM),
            pl.BlockSpec(memory_space=pltpu.VMEM),
        ],
        out_specs=pl.BlockSpec(memory_space=pltpu.VMEM),
        scratch_shapes=[
            pltpu.VMEM((NRINGS, S, CH, SUBW), jnp.bfloat16),
            pltpu.SemaphoreType.DMA((NRINGS, S)),
            pltpu.SemaphoreType.DMA((NRINGS, S)),
            pltpu.SemaphoreType.REGULAR((NRINGS,)),
        ],
        compiler_params=pltpu.CompilerParams(collective_id=0),
    )(x, w_mat)


# device time: 191532 ns/iter; 2.5398x vs baseline; 1.9269x over previous
import jax
import jax.numpy as jnp
from jax import lax
from jax.experimental import pallas as pl
from jax.experimental.pallas import tpu as pltpu

N_DEV = 32
M = 4096
N = 2048
CH = M // N_DEV
NH = N // 2
R = 4
S = 4
SUBW = NH // R
NRINGS = 2 * R
NSTEPS = 2 * (N_DEV - 1)

_GELU_C = 0.7978845608028654


def _gelu(y):
    return 0.5 * y * (1.0 + jnp.tanh(_GELU_C * (y + 0.044715 * y * y * y)))


def kernel(x, w_mat):
    x = x.astype(jnp.bfloat16)
    w_mat = w_mat.astype(jnp.bfloat16)

    def body(x_ref, w_ref, out_ref, comm, ssem, rsem, credit):
        my = lax.axis_index("i")

        def _coords_of_mesh(m):
            z = m // 8
            rr = lax.rem(m, 8)
            y = rr // 2
            xs = lax.rem(rr, 2)
            x = jnp.where(lax.rem(y, 2) == 0, xs, 1 - xs)
            return x, y, z

        def _pos_of_coords(x, y, z):
            q = jnp.where(lax.rem(z, 2) == 0, y, 3 - y)
            p0 = z * 4 + q
            return jnp.where(x == 0, p0, 31 - p0)

        def _coords_of_pos(p):
            p0 = jnp.where(p < 16, p, 31 - p)
            x = jnp.where(p < 16, 0, 1)
            z = p0 // 4
            q = lax.rem(p0, 4)
            y = jnp.where(lax.rem(z, 2) == 0, q, 3 - q)
            return x, y, z

        def _mesh_of_coords(x, y, z):
            return z * 8 + y * 2 + jnp.where(lax.rem(y, 2) == 0, x, 1 - x)

        IDENTITY = False
        if IDENTITY:
            pos = my
            right = lax.rem(my + 1, N_DEV)
            left = lax.rem(my + N_DEV - 1, N_DEV)
        else:
            pos = _pos_of_coords(*_coords_of_mesh(my))
            right = _mesh_of_coords(*_coords_of_pos(lax.rem(pos + 1, N_DEV)))
            left = _mesh_of_coords(*_coords_of_pos(lax.rem(pos + N_DEV - 1, N_DEV)))

        rings = []
        for r in range(R):
            rings.append((2 * r, True, r * SUBW))
            rings.append((2 * r + 1, False, NH + r * SUBW))

        barrier_sem = pltpu.get_barrier_semaphore()
        pl.semaphore_signal(barrier_sem, inc=1, device_id=(left,),
                            device_id_type=pl.DeviceIdType.MESH)
        pl.semaphore_signal(barrier_sem, inc=1, device_id=(right,),
                            device_id_type=pl.DeviceIdType.MESH)
        pl.semaphore_wait(barrier_sem, 2)

        def partial_dir(c, col0):
            xa = x_ref[pl.ds(c * CH, CH), :]
            wa = w_ref[:, pl.ds(col0, NH)]
            return jnp.dot(xa, wa, preferred_element_type=jnp.float32)

        def desc(i, plus, h):
            return pltpu.make_async_remote_copy(
                src_ref=comm.at[i, h % S],
                dst_ref=comm.at[i, (h + 1) % S],
                send_sem=ssem.at[i, h % S],
                recv_sem=rsem.at[i, (h + 1) % S],
                device_id=(right if plus else left,),
                device_id_type=pl.DeviceIdType.MESH,
            )

        p0 = partial_dir(pos, 0).astype(jnp.bfloat16)
        m0 = partial_dir(pos, NH).astype(jnp.bfloat16)
        for i, plus, colbase in rings:
            src = p0 if plus else m0
            cb = colbase if plus else colbase - NH
            comm[i, 0] = src[:, cb:cb + SUBW]
            desc(i, plus, 0).start()

        for h in range(NSTEPS):
            if h <= N_DEV - 2:
                c_p = lax.rem(pos + N_DEV - h - 1, N_DEV)
                c_m = lax.rem(pos + h + 1, N_DEV)
                P_p = partial_dir(c_p, 0)
                P_m = partial_dir(c_m, NH)
            for i, plus, colbase in rings:
                d = desc(i, plus, h)
                slot = (h + 1) % S
                d.wait_recv()
                if h < N_DEV - 2:
                    P = P_p if plus else P_m
                    cb = colbase if plus else colbase - NH
                    summed = (comm[i, slot].astype(jnp.float32)
                              + P[:, cb:cb + SUBW])
                    comm[i, slot] = summed.astype(jnp.bfloat16)
                elif h == N_DEV - 2:
                    P = P_p if plus else P_m
                    cb = colbase if plus else colbase - NH
                    summed = (comm[i, slot].astype(jnp.float32)
                              + P[:, cb:cb + SUBW])
                    g = _gelu(summed).astype(jnp.bfloat16)
                    comm[i, slot] = g
                    own = lax.rem(pos + 1, N_DEV) if plus else lax.rem(pos + N_DEV - 1, N_DEV)
                    out_ref[pl.ds(own * CH, CH), pl.ds(colbase, SUBW)] = g
                if h < NSTEPS - 1:
                    k = h + 1
                    if k >= S - 1:
                        pl.semaphore_wait(credit.at[i], 1)
                    desc(i, plus, k).start()
                if h > N_DEV - 2:
                    t = h - (N_DEV - 1)
                    c = lax.rem(pos + N_DEV - t, N_DEV) if plus else lax.rem(pos + t, N_DEV)
                    out_ref[pl.ds(c * CH, CH), pl.ds(colbase, SUBW)] = comm[i, slot]
                d.wait_send()
                if h <= NSTEPS - 1 - (S - 1):
                    pl.semaphore_signal(
                        credit.at[i], inc=1,
                        device_id=(left if plus else right,),
                        device_id_type=pl.DeviceIdType.MESH,
                    )

    return pl.pallas_call(
        body,
        out_shape=jax.ShapeDtypeStruct((M, N), jnp.bfloat16),
        in_specs=[
            pl.BlockSpec(memory_space=pltpu.VMEM),
            pl.BlockSpec(memory_space=pltpu.VMEM),
        ],
        out_specs=pl.BlockSpec(memory_space=pltpu.VMEM),
        scratch_shapes=[
            pltpu.VMEM((NRINGS, S, CH, SUBW), jnp.bfloat16),
            pltpu.SemaphoreType.DMA((NRINGS, S)),
            pltpu.SemaphoreType.DMA((NRINGS, S)),
            pltpu.SemaphoreType.REGULAR((NRINGS,)),
        ],
        compiler_params=pltpu.CompilerParams(collective_id=2),
    )(x, w_mat)
